# baseline (speedup 1.0000x reference)
import functools

import jax
import jax.numpy as jnp
from jax import lax
from jax.experimental import pallas as pl
from jax.experimental.pallas import tpu as pltpu
from jax.experimental.pallas import tpu_sc as plsc


def _sc_gather(table, idx):
    B = idx.shape[0]
    D = table.shape[1]
    info = plsc.get_sparse_core_info()
    nc, ns = info.num_cores, info.num_subcores
    nw = nc * ns
    b_per_w = B // nw
    mesh = plsc.VectorSubcoreMesh(core_axis_name="c", subcore_axis_name="s")

    @functools.partial(
        pl.kernel,
        mesh=mesh,
        out_type=jax.ShapeDtypeStruct((B, D), jnp.float32),
        scratch_types=[
            pltpu.VMEM((b_per_w,), jnp.int32),
            pltpu.VMEM((b_per_w, D), jnp.float32),
            pltpu.SemaphoreType.DMA,
        ],
        compiler_params=pltpu.CompilerParams(use_tc_tiling_on_sc=False),
    )
    def k(table_hbm, idx_hbm, out_hbm, idx_v, rows_v, sem):
        wid = lax.axis_index("s") * nc + lax.axis_index("c")
        base = wid * b_per_w
        pltpu.sync_copy(idx_hbm.at[pl.ds(base, b_per_w)], idx_v)
        pltpu.async_copy(table_hbm.at[idx_v], rows_v, sem).wait()
        pltpu.sync_copy(rows_v, out_hbm.at[pl.ds(base, b_per_w)])

    return k(table, idx)


def _tc_body(t_ref, q_ref, df_ref, S_ref, Wd_ref, bd_ref, W1q_ref, W1m_ref,
             W1a_ref, w1p_ref, b1_ref, W2_ref, b2_ref, e_ref, p_ref):
    mask = (t_ref[...] != 0).astype(jnp.float32)
    cnt = jnp.maximum(jnp.sum(mask, axis=1, keepdims=True), 1.0)
    mu = lax.dot_general(mask, S_ref[...], (((1,), (0,)), ((), ())),
                         preferred_element_type=jnp.float32) / cnt
    q = q_ref[...]
    a = jnp.dot(df_ref[...], Wd_ref[...],
                preferred_element_type=jnp.float32) + bd_ref[...]
    p12 = jnp.sum(q * mu, axis=-1, keepdims=True)
    p13 = jnp.sum(q * a, axis=-1, keepdims=True)
    p23 = jnp.sum(mu * a, axis=-1, keepdims=True)
    z = (jnp.dot(q, W1q_ref[...], preferred_element_type=jnp.float32)
         + jnp.dot(mu, W1m_ref[...], preferred_element_type=jnp.float32)
         + jnp.dot(a, W1a_ref[...], preferred_element_type=jnp.float32)
         + p12 * w1p_ref[0:1, :] + p13 * w1p_ref[1:2, :] + p23 * w1p_ref[2:3, :]
         + b1_ref[...])
    e = jnp.maximum(z, 0.0)
    e_ref[...] = e
    p_ref[...] = jnp.dot(e, W2_ref[...],
                         preferred_element_type=jnp.float32) + b2_ref[...]


def kernel(questions, question_skill_targets, difficulty_feats, Q_table,
           S_table, W_diff, b_diff, W1, b1, W2, b2):
    B, NT = question_skill_targets.shape
    D = Q_table.shape[1]
    H = W1.shape[1]
    q = _sc_gather(Q_table, questions.astype(jnp.int32))

    bB = 512
    grid = (B // bB,)
    W1q, W1m, W1a, w1p = W1[0:D], W1[D:2 * D], W1[2 * D:3 * D], W1[3 * D:]
    bd2 = b_diff.reshape(1, D)
    b12 = b1.reshape(1, H)
    b22 = b2.reshape(1, 1)

    e, p = pl.pallas_call(
        _tc_body,
        grid=grid,
        in_specs=[
            pl.BlockSpec((bB, NT), lambda i: (i, 0)),
            pl.BlockSpec((bB, D), lambda i: (i, 0)),
            pl.BlockSpec((bB, difficulty_feats.shape[1]), lambda i: (i, 0)),
            pl.BlockSpec((NT, D), lambda i: (0, 0)),
            pl.BlockSpec(W_diff.shape, lambda i: (0, 0)),
            pl.BlockSpec((1, D), lambda i: (0, 0)),
            pl.BlockSpec((D, H), lambda i: (0, 0)),
            pl.BlockSpec((D, H), lambda i: (0, 0)),
            pl.BlockSpec((D, H), lambda i: (0, 0)),
            pl.BlockSpec((3, H), lambda i: (0, 0)),
            pl.BlockSpec((1, H), lambda i: (0, 0)),
            pl.BlockSpec((H, 1), lambda i: (0, 0)),
            pl.BlockSpec((1, 1), lambda i: (0, 0)),
        ],
        out_specs=[
            pl.BlockSpec((bB, H), lambda i: (i, 0)),
            pl.BlockSpec((bB, 1), lambda i: (i, 0)),
        ],
        out_shape=[
            jax.ShapeDtypeStruct((B, H), jnp.float32),
            jax.ShapeDtypeStruct((B, 1), jnp.float32),
        ],
        compiler_params=pltpu.CompilerParams(
            dimension_semantics=("arbitrary",),
        ),
    )(question_skill_targets, q, difficulty_feats, S_table, W_diff, bd2,
      W1q, W1m, W1a, w1p, b12, W2, b22)
    return (e, p)


# speedup vs baseline: 1.5333x; 1.5333x over previous
import functools

import jax
import jax.numpy as jnp
from jax import lax
from jax.experimental import pallas as pl
from jax.experimental.pallas import tpu as pltpu
from jax.experimental.pallas import tpu_sc as plsc


def _sc_gather(table, idx):
    NQ, D = table.shape
    B = idx.shape[0]
    info = plsc.get_sparse_core_info()
    nc, ns, L = info.num_cores, info.num_subcores, info.num_lanes
    nw = nc * ns
    n = B // nw
    K = 16
    mesh = plsc.VectorSubcoreMesh(core_axis_name="c", subcore_axis_name="s")

    @functools.partial(
        pl.kernel,
        mesh=mesh,
        out_type=jax.ShapeDtypeStruct((B, D), jnp.float32),
        scratch_types=[
            pltpu.VMEM((n,), jnp.int32),
            pltpu.VMEM((K, 8, D), jnp.float32),
            pltpu.VMEM((n, D), jnp.float32),
            pltpu.SemaphoreType.DMA,
        ],
        compiler_params=pltpu.CompilerParams(needs_layout_passes=False),
    )
    def k(table_hbm, idx_hbm, out_hbm, idx_v, slabs, rows, sem):
        wid = lax.axis_index("s") * nc + lax.axis_index("c")
        base = wid * n
        pltpu.sync_copy(idx_hbm.at[pl.ds(base, n)], idx_v)

        def chunk(g, _):
            r0 = g * K
            qv = idx_v[pl.ds(r0, L)]
            slabv = jnp.bitwise_and(qv, jnp.int32(~7))
            subv = jnp.bitwise_and(qv, 7)
            for j in range(K):
                slab = pl.multiple_of(slabv[j], 8)
                pltpu.async_copy(table_hbm.at[pl.ds(slab, 8)], slabs.at[j],
                                 sem)
            for j in range(K):
                pltpu.make_async_copy(table_hbm.at[pl.ds(0, 8)], slabs.at[j],
                                      sem).wait()
            for j in range(K):
                sub = subv[j]
                for c in range(D // L):
                    rows[r0 + j, pl.ds(c * L, L)] = slabs[j, sub,
                                                          pl.ds(c * L, L)]
            return 0

        lax.fori_loop(0, n // K, chunk, 0)
        pltpu.sync_copy(rows, out_hbm.at[pl.ds(base, n)])

    return k(table, idx)


def _tc_body(t_ref, q_ref, df_ref, S_ref, Wd_ref, bd_ref, W1q_ref, W1m_ref,
             W1a_ref, w1p_ref, b1_ref, W2_ref, b2_ref, e_ref, p_ref):
    mask = (t_ref[...] != 0).astype(jnp.float32)
    cnt = jnp.maximum(jnp.sum(mask, axis=1, keepdims=True), 1.0)
    mu = lax.dot_general(mask, S_ref[...], (((1,), (0,)), ((), ())),
                         preferred_element_type=jnp.float32) / cnt
    q = q_ref[...]
    a = jnp.dot(df_ref[...], Wd_ref[...],
                preferred_element_type=jnp.float32) + bd_ref[...]
    p12 = jnp.sum(q * mu, axis=-1, keepdims=True)
    p13 = jnp.sum(q * a, axis=-1, keepdims=True)
    p23 = jnp.sum(mu * a, axis=-1, keepdims=True)
    z = (jnp.dot(q, W1q_ref[...], preferred_element_type=jnp.float32)
         + jnp.dot(mu, W1m_ref[...], preferred_element_type=jnp.float32)
         + jnp.dot(a, W1a_ref[...], preferred_element_type=jnp.float32)
         + p12 * w1p_ref[0:1, :] + p13 * w1p_ref[1:2, :] + p23 * w1p_ref[2:3, :]
         + b1_ref[...])
    e = jnp.maximum(z, 0.0)
    e_ref[...] = e
    p_ref[...] = jnp.dot(e, W2_ref[...],
                         preferred_element_type=jnp.float32) + b2_ref[...]


def kernel(questions, question_skill_targets, difficulty_feats, Q_table,
           S_table, W_diff, b_diff, W1, b1, W2, b2):
    B, NT = question_skill_targets.shape
    D = Q_table.shape[1]
    H = W1.shape[1]
    q = _sc_gather(Q_table, questions.astype(jnp.int32))

    bB = 512
    grid = (B // bB,)
    W1q, W1m, W1a, w1p = W1[0:D], W1[D:2 * D], W1[2 * D:3 * D], W1[3 * D:]
    bd2 = b_diff.reshape(1, D)
    b12 = b1.reshape(1, H)
    b22 = b2.reshape(1, 1)

    e, p = pl.pallas_call(
        _tc_body,
        grid=grid,
        in_specs=[
            pl.BlockSpec((bB, NT), lambda i: (i, 0)),
            pl.BlockSpec((bB, D), lambda i: (i, 0)),
            pl.BlockSpec((bB, difficulty_feats.shape[1]), lambda i: (i, 0)),
            pl.BlockSpec((NT, D), lambda i: (0, 0)),
            pl.BlockSpec(W_diff.shape, lambda i: (0, 0)),
            pl.BlockSpec((1, D), lambda i: (0, 0)),
            pl.BlockSpec((D, H), lambda i: (0, 0)),
            pl.BlockSpec((D, H), lambda i: (0, 0)),
            pl.BlockSpec((D, H), lambda i: (0, 0)),
            pl.BlockSpec((3, H), lambda i: (0, 0)),
            pl.BlockSpec((1, H), lambda i: (0, 0)),
            pl.BlockSpec((H, 1), lambda i: (0, 0)),
            pl.BlockSpec((1, 1), lambda i: (0, 0)),
        ],
        out_specs=[
            pl.BlockSpec((bB, H), lambda i: (i, 0)),
            pl.BlockSpec((bB, 1), lambda i: (i, 0)),
        ],
        out_shape=[
            jax.ShapeDtypeStruct((B, H), jnp.float32),
            jax.ShapeDtypeStruct((B, 1), jnp.float32),
        ],
        compiler_params=pltpu.CompilerParams(
            dimension_semantics=("arbitrary",),
        ),
    )(question_skill_targets, q, difficulty_feats, S_table, W_diff, bd2,
      W1q, W1m, W1a, w1p, b12, W2, b22)
    return (e, p)


# speedup vs baseline: 2.0444x; 1.3334x over previous
import functools

import jax
import jax.numpy as jnp
from jax import lax
from jax.experimental import pallas as pl
from jax.experimental.pallas import tpu as pltpu
from jax.experimental.pallas import tpu_sc as plsc


def _sc_gather(table, idx):
    NQ, D = table.shape
    B = idx.shape[0]
    table3 = table.reshape(NQ // 8, 8, D)
    info = plsc.get_sparse_core_info()
    nc, ns, L = info.num_cores, info.num_subcores, info.num_lanes
    nw = nc * ns
    n = B // nw
    K = 16
    mesh = plsc.VectorSubcoreMesh(core_axis_name="c", subcore_axis_name="s")

    @functools.partial(
        pl.kernel,
        mesh=mesh,
        out_type=jax.ShapeDtypeStruct((B, D), jnp.float32),
        scratch_types=[
            pltpu.VMEM((n,), jnp.int32),
            pltpu.VMEM((K, 8, D), jnp.float32),
            pltpu.VMEM((n, D), jnp.float32),
            pltpu.SemaphoreType.DMA,
        ],
        compiler_params=pltpu.CompilerParams(needs_layout_passes=False),
    )
    def k(table_hbm, idx_hbm, out_hbm, idx_v, slabs, rows, sem):
        wid = lax.axis_index("s") * nc + lax.axis_index("c")
        base = wid * n
        pltpu.sync_copy(idx_hbm.at[pl.ds(base, n)], idx_v)

        def chunk(g, _):
            r0 = g * K
            qv = idx_v[pl.ds(r0, L)]
            slabv = jnp.right_shift(qv, 3)
            subv = jnp.bitwise_and(qv, 7)
            for j in range(K):
                pltpu.async_copy(table_hbm.at[slabv[j]], slabs.at[j], sem)
            for j in range(K):
                pltpu.make_async_copy(table_hbm.at[0], slabs.at[j],
                                      sem).wait()
            for j in range(K):
                sub = subv[j]
                for c in range(D // L):
                    rows[r0 + j, pl.ds(c * L, L)] = slabs[j, sub,
                                                          pl.ds(c * L, L)]
            return 0

        lax.fori_loop(0, n // K, chunk, 0)
        pltpu.sync_copy(rows, out_hbm.at[pl.ds(base, n)])

    return k(table3, idx)


def _tc_body(t_ref, q_ref, df_ref, S_ref, Wd_ref, bd_ref, W1q_ref, W1m_ref,
             W1a_ref, w1p_ref, b1_ref, W2_ref, b2_ref, e_ref, p_ref):
    mask = (t_ref[...] != 0).astype(jnp.float32)
    cnt = jnp.maximum(jnp.sum(mask, axis=1, keepdims=True), 1.0)
    mu = lax.dot_general(mask, S_ref[...], (((1,), (0,)), ((), ())),
                         preferred_element_type=jnp.float32) / cnt
    q = q_ref[...]
    a = jnp.dot(df_ref[...], Wd_ref[...],
                preferred_element_type=jnp.float32) + bd_ref[...]
    p12 = jnp.sum(q * mu, axis=-1, keepdims=True)
    p13 = jnp.sum(q * a, axis=-1, keepdims=True)
    p23 = jnp.sum(mu * a, axis=-1, keepdims=True)
    z = (jnp.dot(q, W1q_ref[...], preferred_element_type=jnp.float32)
         + jnp.dot(mu, W1m_ref[...], preferred_element_type=jnp.float32)
         + jnp.dot(a, W1a_ref[...], preferred_element_type=jnp.float32)
         + p12 * w1p_ref[0:1, :] + p13 * w1p_ref[1:2, :] + p23 * w1p_ref[2:3, :]
         + b1_ref[...])
    e = jnp.maximum(z, 0.0)
    e_ref[...] = e
    p_ref[...] = jnp.dot(e, W2_ref[...],
                         preferred_element_type=jnp.float32) + b2_ref[...]


def kernel(questions, question_skill_targets, difficulty_feats, Q_table,
           S_table, W_diff, b_diff, W1, b1, W2, b2):
    B, NT = question_skill_targets.shape
    D = Q_table.shape[1]
    H = W1.shape[1]
    q = _sc_gather(Q_table, questions.astype(jnp.int32))

    bB = 512
    grid = (B // bB,)
    W1q, W1m, W1a, w1p = W1[0:D], W1[D:2 * D], W1[2 * D:3 * D], W1[3 * D:]
    bd2 = b_diff.reshape(1, D)
    b12 = b1.reshape(1, H)
    b22 = b2.reshape(1, 1)

    e, p = pl.pallas_call(
        _tc_body,
        grid=grid,
        in_specs=[
            pl.BlockSpec((bB, NT), lambda i: (i, 0)),
            pl.BlockSpec((bB, D), lambda i: (i, 0)),
            pl.BlockSpec((bB, difficulty_feats.shape[1]), lambda i: (i, 0)),
            pl.BlockSpec((NT, D), lambda i: (0, 0)),
            pl.BlockSpec(W_diff.shape, lambda i: (0, 0)),
            pl.BlockSpec((1, D), lambda i: (0, 0)),
            pl.BlockSpec((D, H), lambda i: (0, 0)),
            pl.BlockSpec((D, H), lambda i: (0, 0)),
            pl.BlockSpec((D, H), lambda i: (0, 0)),
            pl.BlockSpec((3, H), lambda i: (0, 0)),
            pl.BlockSpec((1, H), lambda i: (0, 0)),
            pl.BlockSpec((H, 1), lambda i: (0, 0)),
            pl.BlockSpec((1, 1), lambda i: (0, 0)),
        ],
        out_specs=[
            pl.BlockSpec((bB, H), lambda i: (i, 0)),
            pl.BlockSpec((bB, 1), lambda i: (i, 0)),
        ],
        out_shape=[
            jax.ShapeDtypeStruct((B, H), jnp.float32),
            jax.ShapeDtypeStruct((B, 1), jnp.float32),
        ],
        compiler_params=pltpu.CompilerParams(
            dimension_semantics=("arbitrary",),
        ),
    )(question_skill_targets, q, difficulty_feats, S_table, W_diff, bd2,
      W1q, W1m, W1a, w1p, b12, W2, b22)
    return (e, p)
